# Initial kernel scaffold; baseline (speedup 1.0000x reference)
import jax
import jax.numpy as jnp
from jax.experimental import pallas as pl


def kernel(x, edge_index, W0, b0, W1, b1, W_e2d, Wd0, bd0, Wd1, bd1):
    raise NotImplementedError("write your pallas kernel here")


# speedup vs baseline: 5.2538x; 5.2538x over previous
import functools

import jax
import jax.numpy as jnp
from jax import lax
from jax.experimental import pallas as pl
from jax.experimental.pallas import tpu as pltpu
from jax.experimental.pallas import tpu_sc as plsc

N = 10000
E = 320000
D = 128
NP = 10240
NC, NS, L = 2, 16, 16
NW = NC * NS
EB = 128
NBAT = E // EB
NR = 10240
RPT = NR // NS

def _sc_mesh():
    return plsc.VectorSubcoreMesh(
        core_axis_name="c", subcore_axis_name="s", num_cores=NC, num_subcores=NS)

_f32 = jnp.float32


def _fill_1d(ref, n, value):
    def body(i, _):
        ref[pl.ds(i * 16, 16)] = jnp.full((16,), value, _f32)
        return 0
    lax.fori_loop(0, n // 16, body, 0)


def _zero_rows(ref, rows):
    def body(i, _):
        r = i // (D // 16)
        j = i % (D // 16)
        ref[r, pl.ds(j * 16, 16)] = jnp.zeros((16,), _f32)
        return 0
    lax.fori_loop(0, rows * (D // 16), body, 0)


@functools.cache
def _degrees_kernel():
    return pl.kernel(
        _degrees_body,
        out_type=jax.ShapeDtypeStruct((NC, NP), _f32),
        mesh=_sc_mesh(),
        scratch_types=[
            pltpu.VMEM((EB,), jnp.int32),
            pltpu.VMEM((EB,), _f32),
            pltpu.VMEM((NP // NS,), _f32),
            pltpu.VMEM_SHARED((NP,), _f32),
        ],
    )


def _degrees(e32):
    return _degrees_kernel()(e32)


def _degrees_body(edge_ref, out_ref, idx_v, ones_v, zbuf_v, acc):
    c = lax.axis_index("c")
    s = lax.axis_index("s")
    seg = NP // NS
    _fill_1d(ones_v, EB, 1.0)
    _fill_1d(zbuf_v, seg, 0.0)
    pltpu.sync_copy(zbuf_v, acc.at[pl.ds(s * seg, seg)])
    plsc.subcore_barrier()
    nfull = NBAT // NS

    def do_batch(bid):
        pltpu.sync_copy(edge_ref.at[c, pl.ds(bid * EB, EB)], idx_v)
        pltpu.sync_copy(ones_v, acc.at[idx_v], add=True)

    def body(t, _):
        do_batch(t * NS + s)
        return 0
    lax.fori_loop(0, nfull, body, 0)

    @pl.when(s < NBAT - nfull * NS)
    def _():
        do_batch(nfull * NS + s)

    plsc.subcore_barrier()
    pltpu.sync_copy(acc.at[pl.ds(s * seg, seg)], out_ref.at[c, pl.ds(s * seg, seg)])


@functools.cache
def _seg1_kernel():
    return pl.kernel(
        _seg1_body,
        out_type=jax.ShapeDtypeStruct((NC, NR, D), _f32),
        mesh=_sc_mesh(),
        scratch_types=[
            pltpu.VMEM((EB,), jnp.int32),
            pltpu.VMEM((EB,), jnp.int32),
            pltpu.VMEM((EB, D), _f32),
            pltpu.VMEM_SHARED((NR, D), _f32),
        ],
    )


def _seg1(h, e32):
    return _seg1_kernel()(h, e32)


def _seg1_body(h_ref, edge_ref, out_ref, src_v, dst_v, rows_v, acc):
    c = lax.axis_index("c")
    s = lax.axis_index("s")
    wid = s * NC + c
    _zero_rows(rows_v, EB)
    for k in range(RPT // EB):
        pltpu.sync_copy(rows_v, acc.at[pl.ds(s * RPT + k * EB, EB)])
    plsc.subcore_barrier()
    nfull = NBAT // NW

    def do_batch(bid):
        eb = bid * EB
        pltpu.sync_copy(edge_ref.at[0, pl.ds(eb, EB)], src_v)
        pltpu.sync_copy(edge_ref.at[1, pl.ds(eb, EB)], dst_v)
        pltpu.sync_copy(h_ref.at[src_v], rows_v)
        pltpu.sync_copy(rows_v, acc.at[dst_v], add=True)

    def body(t, _):
        do_batch(t * NW + wid)
        return 0
    lax.fori_loop(0, nfull, body, 0)

    @pl.when(wid < NBAT - nfull * NW)
    def _():
        do_batch(nfull * NW + wid)

    plsc.subcore_barrier()
    pltpu.sync_copy(acc.at[pl.ds(s * RPT, RPT)], out_ref.at[c, pl.ds(s * RPT, RPT)])


@functools.cache
def _seg2_kernel():
    return pl.kernel(
        _seg2_body,
        out_type=jax.ShapeDtypeStruct((NC, NR, D), _f32),
        mesh=_sc_mesh(),
        scratch_types=[
            pltpu.VMEM((EB,), jnp.int32),
            pltpu.VMEM((EB,), jnp.int32),
            pltpu.VMEM((EB, D), _f32),
            pltpu.VMEM_SHARED((NR, D), _f32),
        ],
    )


def _seg2(h2, e32):
    return _seg2_kernel()(h2, e32)


def _seg2_body(h2_ref, edge_ref, out_ref, src_v, dst_v, rows_v, acc):
    c = lax.axis_index("c")
    s = lax.axis_index("s")
    _zero_rows(rows_v, EB)
    for k in range(RPT // EB):
        pltpu.sync_copy(rows_v, acc.at[pl.ds(s * RPT + k * EB, EB)])
    plsc.subcore_barrier()
    nfull = NBAT // NS

    def do_batch(bid):
        eb = bid * EB
        pltpu.sync_copy(edge_ref.at[0, pl.ds(eb, EB)], src_v)
        pltpu.sync_copy(edge_ref.at[1, pl.ds(eb, EB)], dst_v)
        pltpu.sync_copy(h2_ref.at[c].at[src_v], rows_v)
        pltpu.sync_copy(rows_v, acc.at[dst_v], add=True)

    def body(t, _):
        do_batch(t * NS + s)
        return 0
    lax.fori_loop(0, nfull, body, 0)

    @pl.when(s < NBAT - nfull * NS)
    def _():
        do_batch(nfull * NS + s)

    plsc.subcore_barrier()
    pltpu.sync_copy(acc.at[pl.ds(s * RPT, RPT)], out_ref.at[c, pl.ds(s * RPT, RPT)])


RB = 1000
GRID = N // RB


def _mm_scale_body(x_ref, w_ref, deg_ref, o_ref):
    inv = lax.rsqrt(jnp.maximum(deg_ref[0], 1.0))
    o_ref[...] = jnp.dot(x_ref[...], w_ref[...],
                         preferred_element_type=_f32) * inv


def _mm_scale(x, W0, degs3):
    return pl.pallas_call(
        _mm_scale_body,
        grid=(GRID,),
        in_specs=[
            pl.BlockSpec((RB, D), lambda i: (i, 0)),
            pl.BlockSpec((D, D), lambda i: (0, 0)),
            pl.BlockSpec((NC, RB, 1), lambda i: (0, i, 0)),
        ],
        out_specs=pl.BlockSpec((RB, D), lambda i: (i, 0)),
        out_shape=jax.ShapeDtypeStruct((N, D), _f32),
    )(x, W0, degs3)


def _mid_body(p_ref, deg_ref, b0_ref, w1_ref, o_ref):
    inv_out = lax.rsqrt(jnp.maximum(deg_ref[0], 1.0))
    inv_in = lax.rsqrt(jnp.maximum(deg_ref[1], 1.0))
    h1 = jnp.maximum((p_ref[0] + p_ref[1]) * inv_in + b0_ref[...], 0.0)
    hn = h1 * inv_out
    o_ref[0] = jnp.dot(hn, w1_ref[:, :D], preferred_element_type=_f32)
    o_ref[1] = jnp.dot(hn, w1_ref[:, D:], preferred_element_type=_f32)


def _mid(agg1, degs3, b0r, W1):
    return pl.pallas_call(
        _mid_body,
        grid=(GRID,),
        in_specs=[
            pl.BlockSpec((NC, RB, D), lambda i: (0, i, 0)),
            pl.BlockSpec((NC, RB, 1), lambda i: (0, i, 0)),
            pl.BlockSpec((1, D), lambda i: (0, 0)),
            pl.BlockSpec((D, 2 * D), lambda i: (0, 0)),
        ],
        out_specs=pl.BlockSpec((NC, RB, D), lambda i: (0, i, 0)),
        out_shape=jax.ShapeDtypeStruct((NC, N, D), _f32),
    )(agg1, degs3, b0r, W1)


def _tail_body(p_ref, deg_ref, b1_ref, eps_ref, x_ref, we_ref, wd0_ref,
               bd0_ref, wd1_ref, bd1_ref, o_ref, acc):
    i = pl.program_id(0)

    @pl.when(i == 0)
    def _():
        acc[0] = 0.0
        acc[1] = 0.0

    inv_in = lax.rsqrt(jnp.maximum(deg_ref[1], 1.0))
    mu = p_ref[0] * inv_in + b1_ref[0]
    logvar = p_ref[1] * inv_in + b1_ref[1]
    z = mu + eps_ref[...] * jnp.exp(0.5 * logvar)
    rep = jnp.dot(z, we_ref[...], preferred_element_type=_f32)
    hdec = jnp.maximum(
        jnp.dot(rep, wd0_ref[...], preferred_element_type=_f32) + bd0_ref[...],
        0.0)
    logits = jnp.dot(hdec, wd1_ref[...], preferred_element_type=_f32) + bd1_ref[...]
    recon = jax.nn.sigmoid(logits)
    p = jnp.clip(recon, 1e-7, 1.0 - 1e-7)
    xb = x_ref[...]
    bce = -jnp.sum(xb * jnp.log(p) + (1.0 - xb) * jnp.log1p(-p))
    klin = jnp.sum(1.0 + logvar - mu * mu - jnp.exp(logvar))
    acc[0] += bce
    acc[1] += klin

    @pl.when(i == pl.num_programs(0) - 1)
    def _():
        o_ref[0, 0] = acc[0]
        o_ref[0, 1] = acc[1]


def _tail(agg2, degs3, b1r, eps, x, W_e2d, Wd0, bd0r, Wd1, bd1r):
    return pl.pallas_call(
        _tail_body,
        grid=(GRID,),
        in_specs=[
            pl.BlockSpec((NC, RB, D), lambda i: (0, i, 0)),
            pl.BlockSpec((NC, RB, 1), lambda i: (0, i, 0)),
            pl.BlockSpec((2, 1, D), lambda i: (0, 0, 0)),
            pl.BlockSpec((RB, D), lambda i: (i, 0)),
            pl.BlockSpec((RB, D), lambda i: (i, 0)),
            pl.BlockSpec((D, D), lambda i: (0, 0)),
            pl.BlockSpec((D, D), lambda i: (0, 0)),
            pl.BlockSpec((1, D), lambda i: (0, 0)),
            pl.BlockSpec((D, D), lambda i: (0, 0)),
            pl.BlockSpec((1, D), lambda i: (0, 0)),
        ],
        out_specs=pl.BlockSpec(memory_space=pltpu.MemorySpace.SMEM),
        out_shape=jax.ShapeDtypeStruct((1, 2), _f32),
        scratch_shapes=[pltpu.SMEM((2,), _f32)],
    )(agg2, degs3, b1r, eps, x, W_e2d, Wd0, bd0r, Wd1, bd1r)


def kernel(x, edge_index, W0, b0, W1, b1, W_e2d, Wd0, bd0, Wd1, bd1):
    e32 = edge_index.astype(jnp.int32)
    degs = _degrees(e32)
    degs3 = degs.reshape(NC, NP, 1)
    h0n = _mm_scale(x, W0, degs3)
    agg1 = _seg1(h0n, e32)
    h2 = _mid(agg1, degs3, b0.reshape(1, D), W1)
    agg2 = _seg2(h2, e32)
    eps = jax.random.normal(jax.random.key(42), (N, D), dtype=_f32)
    sums = _tail(agg2, degs3, b1.reshape(2, 1, D), eps, x,
                 W_e2d, Wd0, bd0.reshape(1, D), Wd1, bd1.reshape(1, D))
    bce_sum = sums[0, 0]
    klin = sums[0, 1]
    recon_loss = bce_sum / N
    kl_loss = -0.5 * klin / N
    err = -kl_loss
    Pk = 0.02 / (1.0 + jnp.exp(err)) + 0.5
    Ik = -0.001 * err
    Wk = jnp.maximum(Pk + Ik, 1e-6)
    return Wk * kl_loss + recon_loss
